# Initial kernel scaffold; baseline (speedup 1.0000x reference)
import functools

import jax
import jax.numpy as jnp
from jax import lax
from jax.experimental import pallas as pl
from jax.experimental.pallas import tpu as pltpu
from jax.experimental.pallas import tpu_sc as plsc

N = 10000
E = 320000
D = 128
HID = 128
G = 64
C = 16

GW = 144
NC = 2
NS = 16
NW = NC * NS
EPW = E // NW
BLK = 80
NBLK = EPW // BLK
L = 16
RPW = N // NS

_HIGH = lax.Precision.HIGHEST


def _bcast_lane(vec, r):
    idx = jnp.full((L, 1), r, dtype=jnp.int32)
    return lax.gather(
        vec, idx,
        lax.GatherDimensionNumbers(
            offset_dims=(), collapsed_slice_dims=(0,), start_index_map=(0,)),
        slice_sizes=(1,),
        mode=lax.GatherScatterMode.PROMISE_IN_BOUNDS)


def _dense_body(with_prev, with_relu, *refs):
    if with_prev:
        (acc0_ref, acc1_ref, bias_ref, w_ref, att2_ref,
         g_ref, asrc_ref, adst_ref) = refs
        acc = acc0_ref[...] + acc1_ref[...]
        den = jnp.maximum(acc[:, D:D + 1], 1e-16)
        f = acc[:, :D] / den + bias_ref[...]
        if with_relu:
            f = jnp.maximum(f, 0.0)
    else:
        (x_ref, w_ref, att2_ref, g_ref, asrc_ref, adst_ref) = refs
        f = x_ref[...]
    h = lax.dot_general(f, w_ref[...], (((1,), (1,)), ((), ())),
                        precision=_HIGH, preferred_element_type=jnp.float32)
    g_ref[:, :D] = h
    col = lax.broadcasted_iota(jnp.int32, (h.shape[0], GW - D), 1)
    g_ref[:, D:] = jnp.where(col == 0, 1.0, 0.0)
    ab = lax.dot_general(h, att2_ref[...], (((1,), (0,)), ((), ())),
                         precision=_HIGH, preferred_element_type=jnp.float32)
    asrc_ref[...] = ab[:, 0:1]
    adst_ref[...] = ab[:, 1:2]


def _dense_stage(f_or_accs, w, att_src, att_dst, bias=None, with_relu=False):
    att2 = jnp.stack([att_src, att_dst], axis=1)
    out_shape = [
        jax.ShapeDtypeStruct((N, GW), jnp.float32),
        jax.ShapeDtypeStruct((N, 1), jnp.float32),
        jax.ShapeDtypeStruct((N, 1), jnp.float32),
    ]
    if bias is None:
        body = functools.partial(_dense_body, False, False)
        args = (f_or_accs, w, att2)
    else:
        body = functools.partial(_dense_body, True, with_relu)
        acc0, acc1 = f_or_accs
        args = (acc0, acc1, bias.reshape(1, HID), w, att2)
    g, asrc, adst = pl.pallas_call(body, out_shape=out_shape)(*args)
    return g, asrc.reshape(N), adst.reshape(N)


def _edge_body(g_hbm, asrc_hbm, adst_hbm, mvec_hbm, src_hbm, dst_hbm, zero_hbm,
               acc_hbm, asv, adv, sidx, didx, rows, mv, acc_sh, sem):
    c = lax.axis_index("c")
    s = lax.axis_index("s")

    pltpu.sync_copy(asrc_hbm, asv)
    pltpu.sync_copy(adst_hbm, adv)
    pltpu.sync_copy(mvec_hbm, mv)

    pltpu.sync_copy(zero_hbm, acc_sh.at[pl.ds(s * RPW, RPW), :])
    plsc.subcore_barrier()

    wid = c * NS + s
    e0 = wid * EPW
    mvec = mv[...]

    @pl.loop(0, NBLK)
    def _(b):
        base = e0 + b * BLK
        pltpu.sync_copy(src_hbm.at[pl.ds(base, BLK)], sidx)
        pltpu.sync_copy(dst_hbm.at[pl.ds(base, BLK)], didx.at[0])
        pltpu.async_copy(g_hbm.at[sidx], rows, sem).wait()
        for j in range(BLK // L):
            si = sidx[pl.ds(j * L, L)]
            di = didx[0, pl.ds(j * L, L)]
            a_s = plsc.load_gather(asv, [si])
            a_d = plsc.load_gather(adv, [di])
            z = a_s + a_d
            e = jnp.maximum(z, 0.2 * z)
            zb = a_d + mvec
            mb = jnp.maximum(zb, 0.2 * zb)
            ex = jnp.exp(e - mb)
            for r in range(L):
                bv = _bcast_lane(ex, r)
                row = j * L + r
                for k in range(GW // L):
                    sl = (row, pl.ds(k * L, L))
                    rows[sl] = rows[sl] * bv
        pltpu.sync_copy(rows, acc_sh.at[didx.at[0]], add=True)

    plsc.subcore_barrier()
    pltpu.sync_copy(acc_sh.at[pl.ds(s * RPW, RPW), :],
                    acc_hbm.at[c, pl.ds(s * RPW, RPW), :])


_edge_kernel = pl.kernel(
    _edge_body,
    out_type=jax.ShapeDtypeStruct((NC, N, GW), jnp.float32),
    mesh=plsc.VectorSubcoreMesh(core_axis_name="c", subcore_axis_name="s"),
    scratch_types=[
        pltpu.VMEM((N,), jnp.float32),
        pltpu.VMEM((N,), jnp.float32),
        pltpu.VMEM((BLK,), jnp.int32),
        pltpu.VMEM((1, BLK), jnp.int32),
        pltpu.VMEM((BLK, GW), jnp.float32),
        pltpu.VMEM((L,), jnp.float32),
        pltpu.VMEM_SHARED((N, GW), jnp.float32),
        pltpu.SemaphoreType.DMA,
    ],
)


def _edge_stage(g, asrc, adst, src, dst, zero_rows):
    m = jnp.max(asrc)
    mvec = jnp.full((L,), m, dtype=jnp.float32)
    return _edge_kernel(g, asrc, adst, mvec, src, dst, zero_rows)


def _head_body(acc0_ref, acc1_ref, bias_ref, batch_ref, wlin_ref, blin_ref,
               out_ref, xmid_ref):
    acc = acc0_ref[...] + acc1_ref[...]
    den = jnp.maximum(acc[:, D:D + 1], 1e-16)
    f = acc[:, :D] / den
    bt = batch_ref[...]
    gids = lax.broadcasted_iota(jnp.int32, (G, N), 0)
    oh = jnp.where(gids == bt, 1.0, 0.0)
    psum = lax.dot_general(oh, f, (((1,), (0,)), ((), ())),
                           precision=_HIGH, preferred_element_type=jnp.float32)
    cnt = jnp.sum(oh, axis=1, keepdims=True)
    pooled = psum / jnp.maximum(cnt, 1.0) + bias_ref[...]
    xmid_ref[...] = pooled
    out_ref[...] = lax.dot_general(
        pooled, wlin_ref[...], (((1,), (1,)), ((), ())),
        precision=_HIGH, preferred_element_type=jnp.float32) + blin_ref[...]


def _head_stage(acc, b3, batch, wlin, blin):
    out_shape = [
        jax.ShapeDtypeStruct((G, C), jnp.float32),
        jax.ShapeDtypeStruct((G, HID), jnp.float32),
    ]
    return pl.pallas_call(_head_body, out_shape=out_shape)(
        acc[0], acc[1], b3.reshape(1, HID), batch.reshape(1, N).astype(jnp.int32),
        wlin, blin.reshape(1, C))


def kernel(x, edge_index, batch, W1, att_src1, att_dst1, b1,
           W2, att_src2, att_dst2, b2, W3, att_src3, att_dst3, b3,
           Wlin, blin):
    src = edge_index[0].astype(jnp.int32)
    dst = edge_index[1].astype(jnp.int32)
    zero_rows = jnp.zeros((RPW, GW), jnp.float32)

    g1, as1, ad1 = _dense_stage(x, W1, att_src1, att_dst1)
    acc1 = _edge_stage(g1, as1, ad1, src, dst, zero_rows)

    g2, as2, ad2 = _dense_stage((acc1[0], acc1[1]), W2, att_src2, att_dst2,
                                bias=b1, with_relu=True)
    acc2 = _edge_stage(g2, as2, ad2, src, dst, zero_rows)

    g3, as3, ad3 = _dense_stage((acc2[0], acc2[1]), W3, att_src3, att_dst3,
                                bias=b2, with_relu=True)
    acc3 = _edge_stage(g3, as3, ad3, src, dst, zero_rows)

    return _head_stage(acc3, b3, batch, Wlin, blin)


# speedup vs baseline: 20.7968x; 20.7968x over previous
import functools

import jax
import jax.numpy as jnp
from jax import lax
from jax.experimental import pallas as pl
from jax.experimental.pallas import tpu as pltpu
from jax.experimental.pallas import tpu_sc as plsc

N = 10000
E = 320000
D = 128
HID = 128
G = 64
C = 16

GW = 144
NC = 2
NS = 16
NW = NC * NS
EPW = E // NW
BLK = 80
NBLK = EPW // BLK
L = 16
RPW = N // NS

_HIGH = lax.Precision.HIGHEST


def _bcast_lane(vec, r):
    idx = jnp.full((L, 1), r, dtype=jnp.int32)
    return lax.gather(
        vec, idx,
        lax.GatherDimensionNumbers(
            offset_dims=(), collapsed_slice_dims=(0,), start_index_map=(0,)),
        slice_sizes=(1,),
        mode=lax.GatherScatterMode.PROMISE_IN_BOUNDS)


def _dense_body(with_prev, with_relu, *refs):
    if with_prev:
        (acc0_ref, acc1_ref, bias_ref, w_ref, att2_ref,
         g_ref, asrc_ref, adst_ref) = refs
        acc = acc0_ref[...] + acc1_ref[...]
        den = jnp.maximum(acc[:, D:D + 1], 1e-16)
        f = acc[:, :D] / den + bias_ref[...]
        if with_relu:
            f = jnp.maximum(f, 0.0)
    else:
        (x_ref, w_ref, att2_ref, g_ref, asrc_ref, adst_ref) = refs
        f = x_ref[...]
    h = lax.dot_general(f, w_ref[...], (((1,), (1,)), ((), ())),
                        precision=_HIGH, preferred_element_type=jnp.float32)
    g_ref[:, :D] = h
    col = lax.broadcasted_iota(jnp.int32, (h.shape[0], GW - D), 1)
    g_ref[:, D:] = jnp.where(col == 0, 1.0, 0.0)
    ab = lax.dot_general(h, att2_ref[...], (((1,), (0,)), ((), ())),
                         precision=_HIGH, preferred_element_type=jnp.float32)
    asrc_ref[...] = ab[:, 0:1]
    adst_ref[...] = ab[:, 1:2]


def _dense_stage(f_or_accs, w, att_src, att_dst, bias=None, with_relu=False):
    att2 = jnp.stack([att_src, att_dst], axis=1)
    out_shape = [
        jax.ShapeDtypeStruct((N, GW), jnp.float32),
        jax.ShapeDtypeStruct((N, 1), jnp.float32),
        jax.ShapeDtypeStruct((N, 1), jnp.float32),
    ]
    if bias is None:
        body = functools.partial(_dense_body, False, False)
        args = (f_or_accs, w, att2)
    else:
        body = functools.partial(_dense_body, True, with_relu)
        acc0, acc1 = f_or_accs
        args = (acc0, acc1, bias.reshape(1, HID), w, att2)
    g, asrc, adst = pl.pallas_call(body, out_shape=out_shape)(*args)
    return g, asrc.reshape(N), adst.reshape(N)


def _edge_body(g_hbm, asrc_hbm, adst_hbm, mvec_hbm, src_hbm, dst_hbm, zero_hbm,
               acc_hbm, asv, adv, sidx, didx, rows, mv, acc_sh, sem):
    c = lax.axis_index("c")
    s = lax.axis_index("s")

    pltpu.sync_copy(asrc_hbm, asv)
    pltpu.sync_copy(adst_hbm, adv)
    pltpu.sync_copy(mvec_hbm, mv)

    pltpu.sync_copy(zero_hbm, acc_sh.at[pl.ds(s * RPW, RPW), :])
    plsc.subcore_barrier()

    wid = c * NS + s
    e0 = wid * EPW
    mvec = mv[...]

    @pl.loop(0, NBLK)
    def _(b):
        base = e0 + b * BLK
        pltpu.sync_copy(src_hbm.at[pl.ds(base, BLK)], sidx)
        pltpu.sync_copy(dst_hbm.at[pl.ds(base, BLK)], didx.at[0])
        pltpu.async_copy(g_hbm.at[sidx], rows, sem).wait()
        for j in range(BLK // L):
            si = sidx[pl.ds(j * L, L)]
            di = didx[0, pl.ds(j * L, L)]
            a_s = plsc.load_gather(asv, [si])
            a_d = plsc.load_gather(adv, [di])
            z = a_s + a_d
            e = jnp.maximum(z, 0.2 * z)
            zb = a_d + mvec
            mb = jnp.maximum(zb, 0.2 * zb)
            ex = jnp.exp(e - mb)
            for r in range(L):
                bv = _bcast_lane(ex, r)
                row = j * L + r
                for k in range(GW // L):
                    sl = (row, pl.ds(k * L, L))
                    rows[sl] = rows[sl] * bv
        pltpu.sync_copy(rows, acc_sh.at[didx.at[0]], add=True)

    plsc.subcore_barrier()
    pltpu.sync_copy(acc_sh.at[pl.ds(s * RPW, RPW), :],
                    acc_hbm.at[c, pl.ds(s * RPW, RPW), :])


_edge_kernel = pl.kernel(
    _edge_body,
    out_type=jax.ShapeDtypeStruct((NC, N, GW), jnp.float32),
    mesh=plsc.VectorSubcoreMesh(core_axis_name="c", subcore_axis_name="s"),
    scratch_types=[
        pltpu.VMEM((N,), jnp.float32),
        pltpu.VMEM((N,), jnp.float32),
        pltpu.VMEM((BLK,), jnp.int32),
        pltpu.VMEM((1, BLK), jnp.int32),
        pltpu.VMEM((BLK, GW), jnp.float32),
        pltpu.VMEM((L,), jnp.float32),
        pltpu.VMEM_SHARED((N, GW), jnp.float32),
        pltpu.SemaphoreType.DMA,
    ],
    compiler_params=pltpu.CompilerParams(use_tc_tiling_on_sc=False,
                                         needs_layout_passes=False),
)


def _edge_stage(g, asrc, adst, src, dst, zero_rows):
    m = jnp.max(asrc)
    mvec = jnp.full((L,), m, dtype=jnp.float32)
    return _edge_kernel(g, asrc, adst, mvec, src, dst, zero_rows)


def _head_body(acc0_ref, acc1_ref, bias_ref, batch_ref, wlin_ref, blin_ref,
               out_ref, xmid_ref):
    acc = acc0_ref[...] + acc1_ref[...]
    den = jnp.maximum(acc[:, D:D + 1], 1e-16)
    f = acc[:, :D] / den
    bt = batch_ref[...]
    gids = lax.broadcasted_iota(jnp.int32, (G, N), 0)
    oh = jnp.where(gids == bt, 1.0, 0.0)
    psum = lax.dot_general(oh, f, (((1,), (0,)), ((), ())),
                           precision=_HIGH, preferred_element_type=jnp.float32)
    cnt = jnp.sum(oh, axis=1, keepdims=True)
    pooled = psum / jnp.maximum(cnt, 1.0) + bias_ref[...]
    xmid_ref[...] = pooled
    out_ref[...] = lax.dot_general(
        pooled, wlin_ref[...], (((1,), (1,)), ((), ())),
        precision=_HIGH, preferred_element_type=jnp.float32) + blin_ref[...]


def _head_stage(acc, b3, batch, wlin, blin):
    out_shape = [
        jax.ShapeDtypeStruct((G, C), jnp.float32),
        jax.ShapeDtypeStruct((G, HID), jnp.float32),
    ]
    return pl.pallas_call(_head_body, out_shape=out_shape)(
        acc[0], acc[1], b3.reshape(1, HID), batch.reshape(1, N).astype(jnp.int32),
        wlin, blin.reshape(1, C))


def kernel(x, edge_index, batch, W1, att_src1, att_dst1, b1,
           W2, att_src2, att_dst2, b2, W3, att_src3, att_dst3, b3,
           Wlin, blin):
    src = edge_index[0].astype(jnp.int32)
    dst = edge_index[1].astype(jnp.int32)
    zero_rows = jnp.zeros((RPW, GW), jnp.float32)

    g1, as1, ad1 = _dense_stage(x, W1, att_src1, att_dst1)
    acc1 = _edge_stage(g1, as1, ad1, src, dst, zero_rows)

    g2, as2, ad2 = _dense_stage((acc1[0], acc1[1]), W2, att_src2, att_dst2,
                                bias=b1, with_relu=True)
    acc2 = _edge_stage(g2, as2, ad2, src, dst, zero_rows)

    g3, as3, ad3 = _dense_stage((acc2[0], acc2[1]), W3, att_src3, att_dst3,
                                bias=b2, with_relu=True)
    acc3 = _edge_stage(g3, as3, ad3, src, dst, zero_rows)

    return _head_stage(acc3, b3, batch, Wlin, blin)
